# Initial kernel scaffold; baseline (speedup 1.0000x reference)
import jax
import jax.numpy as jnp
from jax.experimental import pallas as pl


def kernel(weights, sharded_sparse_features, sharded_offsets):
    raise NotImplementedError("write your pallas kernel here")


# speedup vs baseline: 6.3261x; 6.3261x over previous
import functools

import jax
import jax.numpy as jnp
from jax import lax
from jax.experimental import pallas as pl
from jax.experimental.pallas import tpu as pltpu
from jax.experimental.pallas import tpu_sc as plsc

B = 1024
L = 20
LP = 24
TD = 26 * 64
LANES = 16

NC = 2
NS = 16
NW = NC * NS
BW = B // NW
NCHUNK = TD // LANES


@functools.lru_cache(maxsize=1)
def _build():
    mesh = plsc.VectorSubcoreMesh(core_axis_name="c", subcore_axis_name="s")

    @functools.partial(
        pl.kernel,
        mesh=mesh,
        out_type=jax.ShapeDtypeStruct((B, TD), jnp.float32),
        scratch_types=[
            pltpu.VMEM((BW, LP), jnp.int32),
            pltpu.VMEM((LP, TD), jnp.float32),
            pltpu.VMEM((TD,), jnp.float32),
            pltpu.SemaphoreType.DMA,
        ],
    )
    def emb_bag(tbl_hbm, idx_hbm, out_hbm, idx_v, rows_v, orow_v, sem):
        wid = lax.axis_index("s") * NC + lax.axis_index("c")
        base = wid * BW
        pltpu.sync_copy(idx_hbm.at[pl.ds(base, BW)], idx_v)

        def bag_body(b, carry):
            pltpu.async_copy(tbl_hbm.at[idx_v.at[b]], rows_v, sem).wait()

            def chunk_body(c, carry2):
                col = pl.ds(c * LANES, LANES)
                vals = [rows_v[r, col] for r in range(L)]
                while len(vals) > 1:
                    nxt = [vals[i] + vals[i + 1] for i in range(0, len(vals) - 1, 2)]
                    if len(vals) % 2:
                        nxt.append(vals[-1])
                    vals = nxt
                orow_v[col] = vals[0]
                return carry2

            lax.fori_loop(0, NCHUNK, chunk_body, 0)
            pltpu.sync_copy(orow_v, out_hbm.at[base + b])
            return carry

        lax.fori_loop(0, BW, bag_body, 0)

    return emb_bag


def kernel(weights, sharded_sparse_features, sharded_offsets):
    del sharded_offsets
    E = weights.shape[0]
    tbl = weights.reshape(E, TD)
    idx = sharded_sparse_features.reshape(B, L)
    idx_pad = jnp.pad(idx, ((0, 0), (0, LP - L)))
    out = _build()(tbl, idx_pad)
    return out.reshape(B, 26, 64)
